# baseline (speedup 1.0000x reference)
import functools

import jax
import jax.numpy as jnp
from jax import lax
from jax.experimental import pallas as pl
from jax.experimental.pallas import tpu as pltpu
from jax.experimental.pallas import tpu_sc as plsc

VOCAB = 100000
HIDDEN = 64
K = 8
MASK_TOKEN_ID = 103
N_ROWS = 512
R = 32
NB = N_ROWS // R
BIG = 2**30


CJ = 200
CW = 500


def _scan_kernel(s_ref, xt_ref, p_ref, tidx_ref, coef_ref):
    p = p_ref[...]
    ent = jnp.sum(
        jnp.where(p > 0, -p * jnp.log(jnp.where(p > 0, p, 1.0)), 0.0),
        axis=(1, 2))

    m_all = jnp.max(p, axis=2)
    ciota = lax.broadcasted_iota(jnp.int32, (R, CJ), 1)
    work_m = m_all
    sels = []
    for _ in range(K):
        m = jnp.max(work_m, axis=1, keepdims=True)
        cand = jnp.where(work_m == m, ciota, BIG)
        sel = jnp.min(cand, axis=1, keepdims=True)
        sels.append(sel)
        work_m = jnp.where(ciota == sel, -1.0, work_m)
    selv = jnp.concatenate(sels, axis=1)

    onehot = (ciota[:, None, :] == selv[..., None]).astype(jnp.float32)
    buf = jnp.stack(
        [jnp.dot(onehot[r], p[r], preferred_element_type=jnp.float32)
         for r in range(R)], axis=0)

    liota = lax.broadcasted_iota(jnp.int32, (R, K, CW), 2)
    gidx = (selv * CW)[..., None] + liota
    vals = []
    idxs = []
    for _ in range(K):
        m = jnp.max(buf, axis=(1, 2), keepdims=True)
        cand = jnp.where(buf == m, gidx, BIG)
        sel = jnp.min(cand, axis=(1, 2), keepdims=True)
        vals.append(m[:, :, 0])
        idxs.append(sel[:, :, 0])
        buf = jnp.where(gidx == sel, -1.0, buf)
    v = jnp.concatenate(vals, axis=1)
    ti = jnp.concatenate(idxs, axis=1)
    wn = v / (jnp.sum(v, axis=1, keepdims=True) + 1e-10)

    s = s_ref[0, 0]
    a = s_ref[0, 1]
    b = s_ref[0, 2]
    inner = a * (-ent - b)
    lam = s * (1.0 / (1.0 + jnp.exp(-inner)))
    xt = xt_ref[0, 0, :]
    lam_m = jnp.where(xt == MASK_TOKEN_ID, lam, 0.0)
    c0 = 1.0 - lam_m
    w = wn * lam_m[:, None]
    coefs = jnp.concatenate(
        [c0[:, None], w, jnp.zeros((R, 7), jnp.float32)], axis=1
    )
    coef_ref[...] = coefs
    tidx_ref[...] = ti


def _tc_scan(scal, xt3, probs2, interpret=False):
    return pl.pallas_call(
        _scan_kernel,
        grid=(NB,),
        in_specs=[
            pl.BlockSpec(memory_space=pltpu.SMEM),
            pl.BlockSpec((1, 1, R), lambda i: (i, 0, 0)),
            pl.BlockSpec((R, CJ, CW), lambda i: (i, 0, 0)),
        ],
        out_specs=[
            pl.BlockSpec((R, K), lambda i: (i, 0)),
            pl.BlockSpec((R, 16), lambda i: (i, 0)),
        ],
        out_shape=[
            jax.ShapeDtypeStruct((N_ROWS, K), jnp.int32),
            jax.ShapeDtypeStruct((N_ROWS, 16), jnp.float32),
        ],
        interpret=interpret,
    )(scal, xt3, probs2)


def _combine_kernel(coef_hbm, tidx_hbm, xt_hbm, table_hbm, out_hbm,
                    tidx_v, xt_v, coef_v, rows8_v, rows1_v, out_v,
                    sem8, sem1):
    wid = lax.axis_index("s") * 2 + lax.axis_index("c")
    base = wid * 16
    pltpu.sync_copy(tidx_hbm.at[pl.ds(wid * 128, 128)], tidx_v)
    pltpu.sync_copy(xt_hbm.at[pl.ds(base, 16)], xt_v)
    pltpu.sync_copy(coef_hbm.at[pl.ds(wid * 256, 256)],
                    coef_v.at[pl.ds(16, 256)])
    cp8 = pltpu.async_copy(table_hbm.at[tidx_v], rows8_v, sem8)
    cp1 = pltpu.async_copy(table_hbm.at[xt_v], rows1_v, sem1)
    cp8.wait()
    cp1.wait()
    for p in range(16):
        c0 = plsc.load_gather(coef_v, [jnp.full((16,), 16 + p * 16, jnp.int32)])
        acc = [c0 * rows1_v[p, pl.ds(h * 16, 16)] for h in range(4)]
        for j in range(K):
            cj = plsc.load_gather(
                coef_v, [jnp.full((16,), 16 + p * 16 + j + 1, jnp.int32)])
            for h in range(4):
                acc[h] = acc[h] + cj * rows8_v[p * K + j, pl.ds(h * 16, 16)]
        for h in range(4):
            out_v[p, pl.ds(h * 16, 16)] = acc[h]
    pltpu.sync_copy(out_v, out_hbm.at[pl.ds(base, 16), :])


def _sc_combine(coef_flat, tidx_flat, xt_flat, table):
    mesh = plsc.VectorSubcoreMesh(core_axis_name="c", subcore_axis_name="s")
    return pl.kernel(
        _combine_kernel,
        out_type=jax.ShapeDtypeStruct((N_ROWS, HIDDEN), jnp.float32),
        mesh=mesh,
        compiler_params=pltpu.CompilerParams(
            needs_layout_passes=False, use_tc_tiling_on_sc=False),
        scratch_types=[
            pltpu.VMEM((128,), jnp.int32),
            pltpu.VMEM((16,), jnp.int32),
            pltpu.VMEM((272,), jnp.float32),
            pltpu.VMEM((128, HIDDEN), jnp.float32),
            pltpu.VMEM((16, HIDDEN), jnp.float32),
            pltpu.VMEM((16, HIDDEN), jnp.float32),
            pltpu.SemaphoreType.DMA,
            pltpu.SemaphoreType.DMA,
        ],
    )(coef_flat, tidx_flat, xt_flat, table)


def kernel(x_t, probs, emb_table, omega_s, omega_a, omega_b):
    batch, seq = x_t.shape
    probs2 = probs.reshape(N_ROWS, CJ, CW)
    xt_flat = x_t.reshape(N_ROWS).astype(jnp.int32)
    xt3 = xt_flat.reshape(NB, 1, R)

    real_s = jax.nn.sigmoid(omega_s)
    real_a = jax.nn.softplus(omega_a)
    real_b = -jax.nn.softplus(omega_b)
    scal = jnp.stack([real_s, real_a, real_b, jnp.float32(0.0)]).reshape(1, 4)

    tidx, coef = _tc_scan(scal, xt3, probs2)
    coef_flat = coef.reshape(N_ROWS * 16)
    tidx_flat = tidx.reshape(N_ROWS * K)

    out = _sc_combine(coef_flat, tidx_flat, xt_flat, emb_table)
    return out.reshape(batch, seq, HIDDEN)


# speedup vs baseline: 2.0578x; 2.0578x over previous
import functools

import jax
import jax.numpy as jnp
from jax import lax
from jax.experimental import pallas as pl
from jax.experimental.pallas import tpu as pltpu
from jax.experimental.pallas import tpu_sc as plsc

VOCAB = 100000
HIDDEN = 64
K = 8
MASK_TOKEN_ID = 103
N_ROWS = 512
R = 32
NB = N_ROWS // R
BIG = 2**30


W = 512
NF = VOCAB // W
TAIL = VOCAB - NF * W
NCH = NF + 1


def _scan_kernel(s_ref, xt_ref, p_ref, tidx_ref, coef_ref):
    p = p_ref[...]
    ent = -jnp.sum(p * jnp.log(p + 1e-38), axis=1)

    pf = p[:, :NF * W].reshape(R, NF, W)
    mf = jnp.max(pf, axis=2)
    mt = jnp.max(p[:, NF * W:], axis=1, keepdims=True)
    m_all = jnp.concatenate([mf, mt], axis=1)

    ciota = lax.broadcasted_iota(jnp.int32, (R, NCH), 1)
    work_m = m_all
    sels = []
    for _ in range(K):
        m = jnp.max(work_m, axis=1, keepdims=True)
        cand = jnp.where(work_m == m, ciota, BIG)
        sel = jnp.min(cand, axis=1, keepdims=True)
        sels.append(sel)
        work_m = jnp.where(ciota == sel, -1.0, work_m)
    selv = jnp.concatenate(sels, axis=1)

    onehot = (ciota[:, None, :] == selv[..., None]).astype(jnp.float32)
    buf = jnp.stack(
        [jnp.dot(onehot[r, :, :NF], pf[r],
                 preferred_element_type=jnp.float32)
         for r in range(R)], axis=0)
    ptail = p[:, VOCAB - W:]
    buf = buf + onehot[:, :, NF:] * ptail[:, None, :]

    liota = lax.broadcasted_iota(jnp.int32, (R, K, W), 2)
    start = jnp.where(selv == NF, VOCAB - W, selv * W)
    start3 = start[..., None]
    gidx = start3 + liota
    dup = (start3 == VOCAB - W) & (liota < W - TAIL)
    buf = jnp.where(dup, -1.0, buf)
    gidx = jnp.where(dup, BIG, gidx)

    vals = []
    idxs = []
    for _ in range(K):
        m = jnp.max(buf, axis=(1, 2), keepdims=True)
        cand = jnp.where(buf == m, gidx, BIG)
        sel = jnp.min(cand, axis=(1, 2), keepdims=True)
        vals.append(m[:, :, 0])
        idxs.append(sel[:, :, 0])
        buf = jnp.where(gidx == sel, -1.0, buf)
    v = jnp.concatenate(vals, axis=1)
    ti = jnp.concatenate(idxs, axis=1)
    wn = v / (jnp.sum(v, axis=1, keepdims=True) + 1e-10)

    s = s_ref[0, 0]
    a = s_ref[0, 1]
    b = s_ref[0, 2]
    inner = a * (-ent - b)
    lam = s * (1.0 / (1.0 + jnp.exp(-inner)))
    xt = xt_ref[0, 0, :]
    lam_m = jnp.where(xt == MASK_TOKEN_ID, lam, 0.0)
    c0 = 1.0 - lam_m
    w = wn * lam_m[:, None]
    coefs = jnp.concatenate(
        [c0[:, None], w, jnp.zeros((R, 7), jnp.float32)], axis=1
    )
    coef_ref[...] = coefs
    tidx_ref[...] = ti


def _tc_scan(scal, xt3, probs2, interpret=False):
    return pl.pallas_call(
        _scan_kernel,
        grid=(NB,),
        in_specs=[
            pl.BlockSpec(memory_space=pltpu.SMEM),
            pl.BlockSpec((1, 1, R), lambda i: (i, 0, 0)),
            pl.BlockSpec((R, VOCAB), lambda i: (i, 0)),
        ],
        out_specs=[
            pl.BlockSpec((R, K), lambda i: (i, 0)),
            pl.BlockSpec((R, 16), lambda i: (i, 0)),
        ],
        out_shape=[
            jax.ShapeDtypeStruct((N_ROWS, K), jnp.int32),
            jax.ShapeDtypeStruct((N_ROWS, 16), jnp.float32),
        ],
        interpret=interpret,
    )(scal, xt3, probs2)


def _combine_kernel(coef_hbm, tidx_hbm, xt_hbm, table_hbm, out_hbm,
                    tidx_v, xt_v, coef_v, rows8_v, rows1_v, out_v,
                    sem8, sem1):
    wid = lax.axis_index("s") * 2 + lax.axis_index("c")
    base = wid * 16
    pltpu.sync_copy(tidx_hbm.at[pl.ds(wid * 128, 128)], tidx_v)
    pltpu.sync_copy(xt_hbm.at[pl.ds(base, 16)], xt_v)
    pltpu.sync_copy(coef_hbm.at[pl.ds(wid * 256, 256)],
                    coef_v.at[pl.ds(16, 256)])
    cp8 = pltpu.async_copy(table_hbm.at[tidx_v], rows8_v, sem8)
    cp1 = pltpu.async_copy(table_hbm.at[xt_v], rows1_v, sem1)
    cp8.wait()
    cp1.wait()
    for p in range(16):
        c0 = plsc.load_gather(coef_v, [jnp.full((16,), 16 + p * 16, jnp.int32)])
        acc = [c0 * rows1_v[p, pl.ds(h * 16, 16)] for h in range(4)]
        for j in range(K):
            cj = plsc.load_gather(
                coef_v, [jnp.full((16,), 16 + p * 16 + j + 1, jnp.int32)])
            for h in range(4):
                acc[h] = acc[h] + cj * rows8_v[p * K + j, pl.ds(h * 16, 16)]
        for h in range(4):
            out_v[p, pl.ds(h * 16, 16)] = acc[h]
    pltpu.sync_copy(out_v, out_hbm.at[pl.ds(base, 16), :])


def _sc_combine(coef_flat, tidx_flat, xt_flat, table):
    mesh = plsc.VectorSubcoreMesh(core_axis_name="c", subcore_axis_name="s")
    return pl.kernel(
        _combine_kernel,
        out_type=jax.ShapeDtypeStruct((N_ROWS, HIDDEN), jnp.float32),
        mesh=mesh,
        compiler_params=pltpu.CompilerParams(
            needs_layout_passes=False, use_tc_tiling_on_sc=False),
        scratch_types=[
            pltpu.VMEM((128,), jnp.int32),
            pltpu.VMEM((16,), jnp.int32),
            pltpu.VMEM((272,), jnp.float32),
            pltpu.VMEM((128, HIDDEN), jnp.float32),
            pltpu.VMEM((16, HIDDEN), jnp.float32),
            pltpu.VMEM((16, HIDDEN), jnp.float32),
            pltpu.SemaphoreType.DMA,
            pltpu.SemaphoreType.DMA,
        ],
    )(coef_flat, tidx_flat, xt_flat, table)


def kernel(x_t, probs, emb_table, omega_s, omega_a, omega_b):
    batch, seq = x_t.shape
    probs2 = probs.reshape(N_ROWS, VOCAB)
    xt_flat = x_t.reshape(N_ROWS).astype(jnp.int32)
    xt3 = xt_flat.reshape(NB, 1, R)

    real_s = jax.nn.sigmoid(omega_s)
    real_a = jax.nn.softplus(omega_a)
    real_b = -jax.nn.softplus(omega_b)
    scal = jnp.stack([real_s, real_a, real_b, jnp.float32(0.0)]).reshape(1, 4)

    tidx, coef = _tc_scan(scal, xt3, probs2)
    coef_flat = coef.reshape(N_ROWS * 16)
    tidx_flat = tidx.reshape(N_ROWS * K)

    out = _sc_combine(coef_flat, tidx_flat, xt_flat, emb_table)
    return out.reshape(batch, seq, HIDDEN)
